# baseline (speedup 1.0000x reference)
import functools

import jax
import jax.numpy as jnp
from jax import lax
from jax.experimental import pallas as pl
from jax.experimental.pallas import tpu as pltpu
from jax.experimental.pallas import tpu_sc as plsc

N = 10000
E = 320000
H = 128
HEADS = 2
VOCAB = 1001
VP = 1008
B = 64
NE = E + N
PE = 331776
NBLK = 1000
NP_DN = 10112
RSL = 2 * NP_DN // 16
K2_EPT = PE // 32
K3_EPT = PE // 16
CHUNK = 128
NCH = K3_EPT // CHUNK
NU = 10240
RPT = NU // 16
CS = 2 * NP_DN


def _k1_body(emb_ref, w_ref, asrc_ref, adst_ref, table_ref, asv_ref, adv_ref):
    th = jnp.dot(emb_ref[...], w_ref[...], preferred_element_type=jnp.float32)
    table_ref[0] = th
    a_s = jnp.sum(th * asrc_ref[0], axis=-1)
    a_d = jnp.sum(th * adst_ref[0], axis=-1)
    asv_ref[0, 0] = a_s
    adv_ref[0, 0] = a_d


def _k1(emb_p, W, att_src3, att_dst3):
    return pl.pallas_call(
        _k1_body,
        grid=(HEADS,),
        in_specs=[
            pl.BlockSpec((VP, H), lambda h: (0, 0)),
            pl.BlockSpec((H, H), lambda h: (0, h)),
            pl.BlockSpec((1, 1, H), lambda h: (h, 0, 0)),
            pl.BlockSpec((1, 1, H), lambda h: (h, 0, 0)),
        ],
        out_specs=[
            pl.BlockSpec((1, VP, H), lambda h: (h, 0, 0)),
            pl.BlockSpec((1, 1, VP), lambda h: (h, 0, 0)),
            pl.BlockSpec((1, 1, VP), lambda h: (h, 0, 0)),
        ],
        out_shape=[
            jax.ShapeDtypeStruct((HEADS, VP, H), jnp.float32),
            jax.ShapeDtypeStruct((HEADS, 1, VP), jnp.float32),
            jax.ShapeDtypeStruct((HEADS, 1, VP), jnp.float32),
        ],
    )(emb_p, W, att_src3, att_dst3)


def _k4_body(u_ref, dn_ref, bi_ref, bias_ref, lng_ref, lnb_ref, linw_ref,
             linb_ref, h_ref, z_ref, zacc, cacc):
    i = pl.program_id(0)
    dn0 = dn_ref[0, 0] + dn_ref[0, 2]
    dn1 = dn_ref[0, 1] + dn_ref[0, 3]
    r0 = u_ref[0] / (dn0[:, None] + 1e-16)
    r1 = u_ref[1] / (dn1[:, None] + 1e-16)
    row = jnp.concatenate([r0, r1], axis=1) + bias_ref[...]
    mu = jnp.mean(row, axis=-1, keepdims=True)
    var = jnp.mean((row - mu) ** 2, axis=-1, keepdims=True)
    xn = (row - mu) * lax.rsqrt(var + 1e-5) * lng_ref[...] + lnb_ref[...]
    y = jnp.dot(xn, linw_ref[...], preferred_element_type=jnp.float32)
    y = y + linb_ref[...]
    hblk = jnp.maximum(y, 0.01 * y)
    h_ref[...] = hblk
    bi = bi_ref[0, 0]
    ob = (bi[:, None] == lax.broadcasted_iota(jnp.int32, (NBLK, B), 1))
    ob = ob.astype(jnp.float32)
    zpart = lax.dot_general(ob, hblk, (((0,), (0,)), ((), ())),
                            preferred_element_type=jnp.float32)
    cpart = jnp.broadcast_to(jnp.sum(ob, axis=0)[:, None], (B, H))

    @pl.when(i == 0)
    def _():
        zacc[...] = jnp.zeros_like(zacc)
        cacc[...] = jnp.zeros_like(cacc)

    zacc[...] += zpart
    cacc[...] += cpart

    @pl.when(i == pl.num_programs(0) - 1)
    def _():
        z_ref[...] = zacc[...] / jnp.maximum(cacc[...], 1.0)


def _k4(U, dnp4, bidx3, bias_gat, ln_g, ln_b, lin_W, lin_b):
    nsteps = N // NBLK
    return pl.pallas_call(
        _k4_body,
        grid=(nsteps,),
        in_specs=[
            pl.BlockSpec((HEADS, NBLK, H), lambda i: (0, i, 0)),
            pl.BlockSpec((1, 2 * HEADS, NBLK), lambda i: (i, 0, 0)),
            pl.BlockSpec((1, 1, NBLK), lambda i: (i, 0, 0)),
            pl.BlockSpec((HEADS * H,), lambda i: (0,)),
            pl.BlockSpec((HEADS * H,), lambda i: (0,)),
            pl.BlockSpec((HEADS * H,), lambda i: (0,)),
            pl.BlockSpec((HEADS * H, H), lambda i: (0, 0)),
            pl.BlockSpec((H,), lambda i: (0,)),
        ],
        out_specs=[
            pl.BlockSpec((NBLK, H), lambda i: (i, 0)),
            pl.BlockSpec((B, H), lambda i: (0, 0)),
        ],
        out_shape=[
            jax.ShapeDtypeStruct((N, H), jnp.float32),
            jax.ShapeDtypeStruct((B, H), jnp.float32),
        ],
        scratch_shapes=[
            pltpu.VMEM((B, H), jnp.float32),
            pltpu.VMEM((B, H), jnp.float32),
        ],
    )(U, dnp4, bidx3, bias_gat, ln_g, ln_b, lin_W, lin_b)


def _k2_body(xf_hbm, src_hbm, dst_hbm, asv_hbm, adv_hbm,
             pack_hbm, dnp_hbm,
             x_v, as0_v, as1_v, ad0_v, ad1_v, srcb, dstb, packb,
             dn0_v, dn1_v, acc_v, tmp_v, tmp2_v, stsem, dnall_sh):
    c = lax.axis_index("c")
    s = lax.axis_index("s")
    wid = s * 2 + c
    base = pl.multiple_of(wid * K2_EPT, 8)
    pltpu.async_copy(xf_hbm, x_v, stsem)
    pltpu.async_copy(asv_hbm.at[pl.ds(0, VP)], as0_v, stsem)
    pltpu.async_copy(asv_hbm.at[pl.ds(VP, VP)], as1_v, stsem)
    pltpu.async_copy(adv_hbm.at[pl.ds(0, VP)], ad0_v, stsem)
    pltpu.async_copy(adv_hbm.at[pl.ds(VP, VP)], ad1_v, stsem)
    pltpu.async_copy(src_hbm.at[pl.ds(base, K2_EPT)], srcb, stsem)
    pltpu.async_copy(dst_hbm.at[pl.ds(base, K2_EPT)], dstb, stsem)

    zero16 = jnp.zeros((16,), jnp.float32)

    def zbody(i, _):
        dn0_v[pl.ds(i * 16, 16)] = zero16
        dn1_v[pl.ds(i * 16, 16)] = zero16
        return 0
    lax.fori_loop(0, NP_DN // 16, zbody, 0)
    for _src, _dst in ((xf_hbm, x_v),
                      (asv_hbm.at[pl.ds(0, VP)], as0_v),
                      (asv_hbm.at[pl.ds(VP, VP)], as1_v),
                      (adv_hbm.at[pl.ds(0, VP)], ad0_v),
                      (adv_hbm.at[pl.ds(VP, VP)], ad1_v),
                      (src_hbm.at[pl.ds(base, K2_EPT)], srcb),
                      (dst_hbm.at[pl.ds(base, K2_EPT)], dstb)):
        pltpu.make_async_copy(_src, _dst, stsem).wait()

    def ebody(ch, _):
        for j in range(CHUNK // 16):
            off = ch * CHUNK + j * 16
            po = ch * (4 * CHUNK) + j * 16
            sv = srcb[pl.ds(off, 16)]
            dv = dstb[pl.ds(off, 16)]
            xs = plsc.load_gather(x_v, [sv])
            xd = plsc.load_gather(x_v, [dv])
            a0 = plsc.load_gather(as0_v, [xs]) + plsc.load_gather(ad0_v, [xd])
            a1 = plsc.load_gather(as1_v, [xs]) + plsc.load_gather(ad1_v, [xd])
            e0 = jnp.exp(jnp.maximum(a0, 0.2 * a0))
            e1 = jnp.exp(jnp.maximum(a1, 0.2 * a1))
            gid = base + off + lax.iota(jnp.int32, 16)
            valid = gid < NE
            e0 = jnp.where(valid, e0, 0.0)
            e1 = jnp.where(valid, e1, 0.0)
            packb[pl.ds(po, 16)] = xs
            packb[pl.ds(po + CHUNK, 16)] = dv
            packb[pl.ds(po + 2 * CHUNK, 16)] = plsc.bitcast(e0, jnp.int32)
            packb[pl.ds(po + 3 * CHUNK, 16)] = plsc.bitcast(e1, jnp.int32)
            plsc.addupdate_scatter(dn0_v, [dv], e0)
            plsc.addupdate_scatter(dn1_v, [dv], e1)
        return 0
    lax.fori_loop(0, K2_EPT // CHUNK, ebody, 0)

    pltpu.sync_copy(packb,
                    pack_hbm.at[pl.ds(pl.multiple_of(base * 4, 8),
                                      4 * K2_EPT)])

    sofs = pl.multiple_of(s * CS, 8)
    pltpu.sync_copy(dn0_v, dnall_sh.at[pl.ds(sofs, NP_DN)])
    pltpu.sync_copy(dn1_v, dnall_sh.at[pl.ds(pl.multiple_of(sofs + NP_DN, 8),
                                             NP_DN)])
    plsc.subcore_barrier()

    sb = pl.multiple_of(s * RSL, 8)
    pltpu.sync_copy(dnall_sh.at[pl.ds(sb, RSL)], acc_v)

    def _partial(j):
        return dnall_sh.at[pl.ds(pl.multiple_of(j * CS + sb, 8), RSL)]

    tmps = (tmp_v, tmp2_v)
    pltpu.async_copy(_partial(1), tmps[1], stsem)
    for j in range(1, 16):
        cur = tmps[j % 2]
        pltpu.make_async_copy(_partial(j), cur, stsem).wait()
        if j < 15:
            pltpu.async_copy(_partial(j + 1), tmps[(j + 1) % 2], stsem)

        def rbody(i, _):
            o = pl.ds(i * 16, 16)
            acc_v[o] = acc_v[o] + cur[o]
            return 0
        lax.fori_loop(0, RSL // 16, rbody, 0)
    pltpu.sync_copy(acc_v, dnp_hbm.at[pl.ds(pl.multiple_of(c * CS + sb, 8),
                                            RSL)])


def _k2(xf, src_p, dst_p, asv2, adv2):
    mesh = plsc.VectorSubcoreMesh(core_axis_name="c", subcore_axis_name="s")
    f = pl.kernel(
        _k2_body,
        out_type=[
            jax.ShapeDtypeStruct((4 * PE,), jnp.int32),
            jax.ShapeDtypeStruct((2 * CS,), jnp.float32),
        ],
        mesh=mesh,
        scratch_types=[
            pltpu.VMEM((N,), jnp.int32),
            pltpu.VMEM((VP,), jnp.float32),
            pltpu.VMEM((VP,), jnp.float32),
            pltpu.VMEM((VP,), jnp.float32),
            pltpu.VMEM((VP,), jnp.float32),
            pltpu.VMEM((K2_EPT,), jnp.int32),
            pltpu.VMEM((K2_EPT,), jnp.int32),
            pltpu.VMEM((4 * K2_EPT,), jnp.int32),
            pltpu.VMEM((NP_DN,), jnp.float32),
            pltpu.VMEM((NP_DN,), jnp.float32),
            pltpu.VMEM((RSL,), jnp.float32),
            pltpu.VMEM((RSL,), jnp.float32),
            pltpu.VMEM((RSL,), jnp.float32),
            pltpu.SemaphoreType.DMA,
            pltpu.VMEM_SHARED((16 * CS,), jnp.float32),
        ],
        compiler_params=pltpu.CompilerParams(needs_layout_passes=False),
    )
    return f(xf, src_p, dst_p, asv2, adv2)


def _k3_body(pack_hbm, table2_hbm,
             u_hbm,
             ipack0, ipack1, dstb0a, dstb0b, dstb1a, dstb1b, grow0, grow1,
             isem0, isem1, gsem0, gsem1, ssem0, ssem1, u_sh):
    c = lax.axis_index("c")
    s = lax.axis_index("s")
    REC = 4 * CHUNK
    tb4 = s * NCH * REC

    zero16 = jnp.zeros((16,), jnp.float32)

    def zb(i, _):
        for j in range(8):
            grow0[i, pl.ds(j * 16, 16)] = zero16
        return 0
    lax.fori_loop(0, CHUNK, zb, 0)
    rs = pl.multiple_of(s * RPT, 8)
    for k in range(RPT // CHUNK):
        pltpu.sync_copy(grow0,
                        u_sh.at[pl.ds(pl.multiple_of(rs + k * CHUNK, 8),
                                      CHUNK)])
    rem = RPT % CHUNK
    if rem:
        pltpu.sync_copy(grow0.at[pl.ds(0, rem)],
                        u_sh.at[pl.ds(pl.multiple_of(rs + RPT - rem, 8),
                                      rem)])
    plsc.subcore_barrier()

    cvp = c * VP
    exoff = 2 * CHUNK + c * CHUNK
    IP = (ipack0, ipack1)
    IS = (isem0, isem1)
    GR = (grow0, grow1)
    DBA = (dstb0a, dstb1a)
    DBB = (dstb0b, dstb1b)
    GS = (gsem0, gsem1)
    SS = (ssem0, ssem1)
    HC = CHUNK // 2

    def stage_idx(g, slot):
        src = pack_hbm.at[pl.ds(pl.multiple_of(tb4 + g * REC, 8), REC)]
        return pltpu.async_copy(src, IP[slot], IS[slot])

    def gat(slot, grow, gsem):
        ip = IP[slot]

        def xsbody(i, _):
            o = pl.ds(i * 16, 16)
            ip[o] = ip[o] + cvp
            return 0
        lax.fori_loop(0, CHUNK // 16, xsbody, 0)
        return pltpu.async_copy(table2_hbm.at[ip.at[pl.ds(0, CHUNK)]],
                                grow, gsem)

    def scale_scat(b):
        ip = IP[b]
        grow = GR[b]

        def half(dstb, lo):
            def scbody(gr, _):
                o16 = lo + gr * 16
                dstb[pl.ds(gr * 16, 16)] = ip[pl.ds(CHUNK + o16, 16)]
                exv = plsc.bitcast(ip[pl.ds(exoff + o16, 16)], jnp.float32)
                for l in range(16):
                    e = lo + gr * 16 + l
                    sc = exv[l]
                    for j in range(8):
                        o = pl.ds(j * 16, 16)
                        grow[e, o] = grow[e, o] * sc
                return 0
            lax.fori_loop(0, HC // 16, scbody, 0)
            pltpu.async_copy(grow.at[pl.ds(lo, HC)], u_sh.at[dstb],
                             SS[b], add=True)
        half(DBA[b], 0)
        half(DBB[b], HC)

    def wsem_i(slot):
        pltpu.make_async_copy(pack_hbm.at[pl.ds(0, REC)], IP[slot],
                              IS[slot]).wait()

    def wsem_g(grow, gsem):
        pltpu.make_async_copy(table2_hbm.at[pl.ds(0, CHUNK)], grow,
                              gsem).wait()

    def wsem_s(b):
        pltpu.make_async_copy(GR[b].at[pl.ds(0, HC)], u_sh.at[DBA[b]],
                              SS[b]).wait()
        pltpu.make_async_copy(GR[b].at[pl.ds(HC, HC)], u_sh.at[DBB[b]],
                              SS[b]).wait()

    def body(b, ssem_pending, do_next, nxt):
        nb = 1 - b
        if do_next:
            wsem_i(nb)
            if ssem_pending:
                wsem_s(nb)
            gat(nb, GR[nb], GS[nb])
        wsem_g(GR[b], GS[b])
        scale_scat(b)
        if do_next:
            stage_idx(nxt, b)

    stage_idx(0, 0)
    stage_idx(1, 1)
    wsem_i(0)
    gat(0, grow0, gsem0)
    body(0, False, True, 2)

    def pair(p, _):
        a = 2 * p + 1
        body(1, True, True, a + 2)
        body(0, True, True, jnp.minimum(a + 3, NCH - 1))
        return 0
    lax.fori_loop(0, (NCH - 2) // 2, pair, 0)
    wsem_s(0)
    body(1, False, False, 0)
    wsem_s(1)
    wsem_i(0)

    plsc.subcore_barrier()
    pltpu.sync_copy(u_sh.at[pl.ds(rs, RPT)], u_hbm.at[c, pl.ds(rs, RPT)])


def _k3(pack, table2):
    mesh = plsc.VectorSubcoreMesh(core_axis_name="c", subcore_axis_name="s")
    f = pl.kernel(
        _k3_body,
        out_type=jax.ShapeDtypeStruct((HEADS, NU, H), jnp.float32),
        mesh=mesh,
        scratch_types=[
            pltpu.VMEM((4 * CHUNK,), jnp.int32),
            pltpu.VMEM((4 * CHUNK,), jnp.int32),
            pltpu.VMEM((CHUNK // 2,), jnp.int32),
            pltpu.VMEM((CHUNK // 2,), jnp.int32),
            pltpu.VMEM((CHUNK // 2,), jnp.int32),
            pltpu.VMEM((CHUNK // 2,), jnp.int32),
            pltpu.VMEM((CHUNK, H), jnp.float32),
            pltpu.VMEM((CHUNK, H), jnp.float32),
            pltpu.SemaphoreType.DMA,
            pltpu.SemaphoreType.DMA,
            pltpu.SemaphoreType.DMA,
            pltpu.SemaphoreType.DMA,
            pltpu.SemaphoreType.DMA,
            pltpu.SemaphoreType.DMA,
            pltpu.VMEM_SHARED((NU, H), jnp.float32),
        ],
        compiler_params=pltpu.CompilerParams(needs_layout_passes=False),
    )
    return f(pack, table2)


def _edge_phase_jax(xf, src, dst, table, asv, adv):
    a_s = asv[:, 0, :][:, xf]
    a_d = adv[:, 0, :][:, xf]
    alpha = a_s[:, src] + a_d[:, dst]
    alpha = jnp.maximum(alpha, 0.2 * alpha)
    ex = jnp.exp(alpha)
    dn = jax.ops.segment_sum(ex.T, dst, num_segments=N)
    xs = xf[src]
    u0 = jax.ops.segment_sum(table[0][xs] * ex[0][:, None], dst, num_segments=N)
    u1 = jax.ops.segment_sum(table[1][xs] * ex[1][:, None], dst, num_segments=N)
    U = jnp.stack([u0, u1], axis=0)
    dnp = jnp.zeros((2, HEADS, N // NBLK, NBLK), jnp.float32)
    dnp = dnp.at[0].set(dn.T.reshape(HEADS, N // NBLK, NBLK))
    dnp = dnp.reshape(2 * HEADS, N // NBLK, NBLK).transpose(1, 0, 2)
    return U, dnp


def kernel(x, edge_index, batch_idx, emb, W, att_src, att_dst, bias_gat,
           ln_g, ln_b, lin_W, lin_b):
    xf = x.reshape(N).astype(jnp.int32)
    loop = jnp.arange(N, dtype=jnp.int32)
    pad = jnp.zeros(PE - NE, dtype=jnp.int32)
    src_p = jnp.concatenate([edge_index[0].astype(jnp.int32), loop, pad])
    dst_p = jnp.concatenate([edge_index[1].astype(jnp.int32), loop, pad])

    emb_p = jnp.pad(emb, ((0, VP - VOCAB), (0, 0)))
    att_src3 = att_src.reshape(HEADS, 1, H)
    att_dst3 = att_dst.reshape(HEADS, 1, H)
    table, asv, adv = _k1(emb_p, W, att_src3, att_dst3)

    asv2 = asv.reshape(HEADS * VP)
    adv2 = adv.reshape(HEADS * VP)
    pack, dnp_flat = _k2(xf, src_p, dst_p, asv2, adv2)
    table2 = table.reshape(HEADS * VP, H)
    U = _k3(pack, table2)

    dnp = dnp_flat.reshape(2, 2, NP_DN)[:, :, :N]
    dnp = dnp.reshape(2 * HEADS, N // NBLK, NBLK).transpose(1, 0, 2)

    bidx3 = batch_idx.astype(jnp.int32).reshape(N // NBLK, 1, NBLK)
    h, z = _k4(U, dnp, bidx3, bias_gat, ln_g, ln_b, lin_W, lin_b)
    return h, z


# speedup vs baseline: 1.0075x; 1.0075x over previous
import jax
import jax.numpy as jnp
from jax import lax
from jax.experimental import pallas as pl
from jax.experimental.pallas import tpu as pltpu
from jax.experimental.pallas import tpu_sc as plsc

N = 10000
E = 320000
H = 128
HEADS = 2
VOCAB = 1001
VP = 1008
B = 64
NE = E + N
PE = 331776
NBLK = 1000
NP_DN = 10112
RSL = 2 * NP_DN // 16
K2_EPT = PE // 32
K3_EPT = PE // 16
CHUNK = 128
NCH = K3_EPT // CHUNK
NU = 10240
RPT = NU // 16
CS = 2 * NP_DN


def _k1_body(emb_ref, w_ref, asrc_ref, adst_ref, table_ref, asv_ref, adv_ref):
    th = jnp.dot(emb_ref[...], w_ref[...], preferred_element_type=jnp.float32)
    table_ref[0] = th
    a_s = jnp.sum(th * asrc_ref[0], axis=-1)
    a_d = jnp.sum(th * adst_ref[0], axis=-1)
    asv_ref[0, 0] = a_s
    adv_ref[0, 0] = a_d


def _k1(emb_p, W, att_src3, att_dst3):
    return pl.pallas_call(
        _k1_body,
        grid=(HEADS,),
        in_specs=[
            pl.BlockSpec((VP, H), lambda h: (0, 0)),
            pl.BlockSpec((H, H), lambda h: (0, h)),
            pl.BlockSpec((1, 1, H), lambda h: (h, 0, 0)),
            pl.BlockSpec((1, 1, H), lambda h: (h, 0, 0)),
        ],
        out_specs=[
            pl.BlockSpec((1, VP, H), lambda h: (h, 0, 0)),
            pl.BlockSpec((1, 1, VP), lambda h: (h, 0, 0)),
            pl.BlockSpec((1, 1, VP), lambda h: (h, 0, 0)),
        ],
        out_shape=[
            jax.ShapeDtypeStruct((HEADS, VP, H), jnp.float32),
            jax.ShapeDtypeStruct((HEADS, 1, VP), jnp.float32),
            jax.ShapeDtypeStruct((HEADS, 1, VP), jnp.float32),
        ],
    )(emb_p, W, att_src3, att_dst3)


def _k4_body(u_ref, dn_ref, bi_ref, bias_ref, lng_ref, lnb_ref, linw_ref,
             linb_ref, h_ref, z_ref, zacc, cacc):
    i = pl.program_id(0)
    dn0 = dn_ref[0, 0] + dn_ref[0, 2]
    dn1 = dn_ref[0, 1] + dn_ref[0, 3]
    r0 = u_ref[0] / (dn0[:, None] + 1e-16)
    r1 = u_ref[1] / (dn1[:, None] + 1e-16)
    row = jnp.concatenate([r0, r1], axis=1) + bias_ref[...]
    mu = jnp.mean(row, axis=-1, keepdims=True)
    var = jnp.mean((row - mu) ** 2, axis=-1, keepdims=True)
    xn = (row - mu) * lax.rsqrt(var + 1e-5) * lng_ref[...] + lnb_ref[...]
    y = jnp.dot(xn, linw_ref[...], preferred_element_type=jnp.float32)
    y = y + linb_ref[...]
    hblk = jnp.maximum(y, 0.01 * y)
    h_ref[...] = hblk
    bi = bi_ref[0, 0]
    ob = (bi[:, None] == lax.broadcasted_iota(jnp.int32, (NBLK, B), 1))
    ob = ob.astype(jnp.float32)
    zpart = lax.dot_general(ob, hblk, (((0,), (0,)), ((), ())),
                            preferred_element_type=jnp.float32)
    cpart = jnp.broadcast_to(jnp.sum(ob, axis=0)[:, None], (B, H))

    @pl.when(i == 0)
    def _():
        zacc[...] = jnp.zeros_like(zacc)
        cacc[...] = jnp.zeros_like(cacc)

    zacc[...] += zpart
    cacc[...] += cpart

    @pl.when(i == pl.num_programs(0) - 1)
    def _():
        z_ref[...] = zacc[...] / jnp.maximum(cacc[...], 1.0)


def _k4(U, dnp4, bidx3, bias_gat, ln_g, ln_b, lin_W, lin_b):
    nsteps = N // NBLK
    return pl.pallas_call(
        _k4_body,
        grid=(nsteps,),
        in_specs=[
            pl.BlockSpec((HEADS, NBLK, H), lambda i: (0, i, 0)),
            pl.BlockSpec((1, 2 * HEADS, NBLK), lambda i: (i, 0, 0)),
            pl.BlockSpec((1, 1, NBLK), lambda i: (i, 0, 0)),
            pl.BlockSpec((HEADS * H,), lambda i: (0,)),
            pl.BlockSpec((HEADS * H,), lambda i: (0,)),
            pl.BlockSpec((HEADS * H,), lambda i: (0,)),
            pl.BlockSpec((HEADS * H, H), lambda i: (0, 0)),
            pl.BlockSpec((H,), lambda i: (0,)),
        ],
        out_specs=[
            pl.BlockSpec((NBLK, H), lambda i: (i, 0)),
            pl.BlockSpec((B, H), lambda i: (0, 0)),
        ],
        out_shape=[
            jax.ShapeDtypeStruct((N, H), jnp.float32),
            jax.ShapeDtypeStruct((B, H), jnp.float32),
        ],
        scratch_shapes=[
            pltpu.VMEM((B, H), jnp.float32),
            pltpu.VMEM((B, H), jnp.float32),
        ],
    )(U, dnp4, bidx3, bias_gat, ln_g, ln_b, lin_W, lin_b)


def _k2_body(xf_hbm, src_hbm, dst_hbm, asv_hbm, adv_hbm,
             pack_hbm, dnp_hbm,
             x_v, as0_v, as1_v, ad0_v, ad1_v, srcb, dstb, packb,
             dn0_v, dn1_v, acc_v, tmp_v, tmp2_v, stsem, dnall_sh):
    c = lax.axis_index("c")
    s = lax.axis_index("s")
    wid = s * 2 + c
    base = pl.multiple_of(wid * K2_EPT, 8)
    pltpu.async_copy(xf_hbm, x_v, stsem)
    pltpu.async_copy(asv_hbm.at[pl.ds(0, VP)], as0_v, stsem)
    pltpu.async_copy(asv_hbm.at[pl.ds(VP, VP)], as1_v, stsem)
    pltpu.async_copy(adv_hbm.at[pl.ds(0, VP)], ad0_v, stsem)
    pltpu.async_copy(adv_hbm.at[pl.ds(VP, VP)], ad1_v, stsem)
    pltpu.async_copy(src_hbm.at[pl.ds(base, K2_EPT)], srcb, stsem)
    pltpu.async_copy(dst_hbm.at[pl.ds(base, K2_EPT)], dstb, stsem)

    zero16 = jnp.zeros((16,), jnp.float32)

    def zbody(i, _):
        dn0_v[pl.ds(i * 16, 16)] = zero16
        dn1_v[pl.ds(i * 16, 16)] = zero16
        return 0
    lax.fori_loop(0, NP_DN // 16, zbody, 0)
    for _src, _dst in ((xf_hbm, x_v),
                      (asv_hbm.at[pl.ds(0, VP)], as0_v),
                      (asv_hbm.at[pl.ds(VP, VP)], as1_v),
                      (adv_hbm.at[pl.ds(0, VP)], ad0_v),
                      (adv_hbm.at[pl.ds(VP, VP)], ad1_v),
                      (src_hbm.at[pl.ds(base, K2_EPT)], srcb),
                      (dst_hbm.at[pl.ds(base, K2_EPT)], dstb)):
        pltpu.make_async_copy(_src, _dst, stsem).wait()

    def ebody(ch, _):
        for j in range(CHUNK // 16):
            off = ch * CHUNK + j * 16
            po = ch * (4 * CHUNK) + j * 16
            sv = srcb[pl.ds(off, 16)]
            dv = dstb[pl.ds(off, 16)]
            xs = plsc.load_gather(x_v, [sv])
            xd = plsc.load_gather(x_v, [dv])
            a0 = plsc.load_gather(as0_v, [xs]) + plsc.load_gather(ad0_v, [xd])
            a1 = plsc.load_gather(as1_v, [xs]) + plsc.load_gather(ad1_v, [xd])
            e0 = jnp.exp(jnp.maximum(a0, 0.2 * a0))
            e1 = jnp.exp(jnp.maximum(a1, 0.2 * a1))
            gid = base + off + lax.iota(jnp.int32, 16)
            valid = gid < NE
            e0 = jnp.where(valid, e0, 0.0)
            e1 = jnp.where(valid, e1, 0.0)
            packb[pl.ds(po, 16)] = xs
            packb[pl.ds(po + CHUNK, 16)] = dv
            packb[pl.ds(po + 2 * CHUNK, 16)] = plsc.bitcast(e0, jnp.int32)
            packb[pl.ds(po + 3 * CHUNK, 16)] = plsc.bitcast(e1, jnp.int32)
            plsc.addupdate_scatter(dn0_v, [dv], e0)
            plsc.addupdate_scatter(dn1_v, [dv], e1)
        return 0
    lax.fori_loop(0, K2_EPT // CHUNK, ebody, 0)

    pltpu.sync_copy(packb,
                    pack_hbm.at[pl.ds(pl.multiple_of(base * 4, 8),
                                      4 * K2_EPT)])

    sofs = pl.multiple_of(s * CS, 8)
    pltpu.sync_copy(dn0_v, dnall_sh.at[pl.ds(sofs, NP_DN)])
    pltpu.sync_copy(dn1_v, dnall_sh.at[pl.ds(pl.multiple_of(sofs + NP_DN, 8),
                                             NP_DN)])
    plsc.subcore_barrier()

    sb = pl.multiple_of(s * RSL, 8)
    pltpu.sync_copy(dnall_sh.at[pl.ds(sb, RSL)], acc_v)

    def _partial(j):
        return dnall_sh.at[pl.ds(pl.multiple_of(j * CS + sb, 8), RSL)]

    tmps = (tmp_v, tmp2_v)
    pltpu.async_copy(_partial(1), tmps[1], stsem)
    for j in range(1, 16):
        cur = tmps[j % 2]
        pltpu.make_async_copy(_partial(j), cur, stsem).wait()
        if j < 15:
            pltpu.async_copy(_partial(j + 1), tmps[(j + 1) % 2], stsem)

        def rbody(i, _):
            o = pl.ds(i * 16, 16)
            acc_v[o] = acc_v[o] + cur[o]
            return 0
        lax.fori_loop(0, RSL // 16, rbody, 0)
    pltpu.sync_copy(acc_v, dnp_hbm.at[pl.ds(pl.multiple_of(c * CS + sb, 8),
                                            RSL)])


def _k2(xf, src_p, dst_p, asv2, adv2):
    mesh = plsc.VectorSubcoreMesh(core_axis_name="c", subcore_axis_name="s")
    f = pl.kernel(
        _k2_body,
        out_type=[
            jax.ShapeDtypeStruct((4 * PE,), jnp.int32),
            jax.ShapeDtypeStruct((2 * CS,), jnp.float32),
        ],
        mesh=mesh,
        scratch_types=[
            pltpu.VMEM((N,), jnp.int32),
            pltpu.VMEM((VP,), jnp.float32),
            pltpu.VMEM((VP,), jnp.float32),
            pltpu.VMEM((VP,), jnp.float32),
            pltpu.VMEM((VP,), jnp.float32),
            pltpu.VMEM((K2_EPT,), jnp.int32),
            pltpu.VMEM((K2_EPT,), jnp.int32),
            pltpu.VMEM((4 * K2_EPT,), jnp.int32),
            pltpu.VMEM((NP_DN,), jnp.float32),
            pltpu.VMEM((NP_DN,), jnp.float32),
            pltpu.VMEM((RSL,), jnp.float32),
            pltpu.VMEM((RSL,), jnp.float32),
            pltpu.VMEM((RSL,), jnp.float32),
            pltpu.SemaphoreType.DMA,
            pltpu.VMEM_SHARED((16 * CS,), jnp.float32),
        ],
        compiler_params=pltpu.CompilerParams(needs_layout_passes=False),
    )
    return f(xf, src_p, dst_p, asv2, adv2)


def _k3_body(pack_hbm, table2_hbm,
             u_hbm,
             ipack0, ipack1, dstb0, dstb1, grow0, grow1,
             isem0, isem1, gsem0, gsem1, ssem0, ssem1, u_sh):
    c = lax.axis_index("c")
    s = lax.axis_index("s")
    REC = 4 * CHUNK
    tb4 = s * NCH * REC

    zero16 = jnp.zeros((16,), jnp.float32)

    def zb(i, _):
        for j in range(8):
            grow0[i, pl.ds(j * 16, 16)] = zero16
        return 0
    lax.fori_loop(0, CHUNK, zb, 0)
    rs = pl.multiple_of(s * RPT, 8)
    for k in range(RPT // CHUNK):
        pltpu.sync_copy(grow0,
                        u_sh.at[pl.ds(pl.multiple_of(rs + k * CHUNK, 8),
                                      CHUNK)])
    rem = RPT % CHUNK
    if rem:
        pltpu.sync_copy(grow0.at[pl.ds(0, rem)],
                        u_sh.at[pl.ds(pl.multiple_of(rs + RPT - rem, 8),
                                      rem)])
    plsc.subcore_barrier()

    cvp = c * VP
    exoff = 2 * CHUNK + c * CHUNK
    IP = (ipack0, ipack1)
    IS = (isem0, isem1)
    GR = (grow0, grow1)
    DB = (dstb0, dstb1)
    GS = (gsem0, gsem1)
    SS = (ssem0, ssem1)

    def stage_idx(g, slot):
        src = pack_hbm.at[pl.ds(pl.multiple_of(tb4 + g * REC, 8), REC)]
        return pltpu.async_copy(src, IP[slot], IS[slot])

    def gat(slot, grow, gsem):
        ip = IP[slot]

        def xsbody(i, _):
            o = pl.ds(i * 16, 16)
            ip[o] = ip[o] + cvp
            return 0
        lax.fori_loop(0, CHUNK // 16, xsbody, 0)
        return pltpu.async_copy(table2_hbm.at[ip.at[pl.ds(0, CHUNK)]],
                                grow, gsem)

    def scale_prep(slot, grow, srow, dstb):
        ip = IP[slot]

        def scbody(gr, _):
            o16 = gr * 16
            dstb[pl.ds(o16, 16)] = ip[pl.ds(CHUNK + o16, 16)]
            exv = plsc.bitcast(ip[pl.ds(exoff + o16, 16)], jnp.float32)
            for l in range(16):
                e = gr * 16 + l
                sc = exv[l]
                for j in range(8):
                    o = pl.ds(j * 16, 16)
                    srow[e, o] = grow[e, o] * sc
            return 0
        lax.fori_loop(0, CHUNK // 16, scbody, 0)

    def scat(srow, dstb, ssem):
        return pltpu.async_copy(srow, u_sh.at[dstb], ssem, add=True)

    def wsem_i(slot):
        pltpu.make_async_copy(pack_hbm.at[pl.ds(0, REC)], IP[slot],
                              IS[slot]).wait()

    def wsem_g(grow, gsem):
        pltpu.make_async_copy(table2_hbm.at[pl.ds(0, CHUNK)], grow,
                              gsem).wait()

    def wsem_s(srow, dstb, ssem):
        pltpu.make_async_copy(srow, u_sh.at[dstb], ssem).wait()

    def body(b, ssem_pending, do_next, nxt):
        nb = 1 - b
        if do_next:
            wsem_i(nb)
            if ssem_pending:
                wsem_s(GR[nb], DB[nb], SS[nb])
            gat(nb, GR[nb], GS[nb])
        wsem_g(GR[b], GS[b])
        scale_prep(b, GR[b], GR[b], DB[b])
        scat(GR[b], DB[b], SS[b])
        if do_next:
            stage_idx(nxt, b)

    stage_idx(0, 0)
    stage_idx(1, 1)
    wsem_i(0)
    gat(0, grow0, gsem0)
    body(0, False, True, 2)

    def pair(p, _):
        a = 2 * p + 1
        body(1, True, True, a + 2)
        body(0, True, True, jnp.minimum(a + 3, NCH - 1))
        return 0
    lax.fori_loop(0, (NCH - 2) // 2, pair, 0)
    wsem_s(grow0, dstb0, ssem0)
    body(1, False, False, 0)
    wsem_s(grow1, dstb1, ssem1)
    wsem_i(0)

    plsc.subcore_barrier()
    pltpu.sync_copy(u_sh.at[pl.ds(rs, RPT)], u_hbm.at[c, pl.ds(rs, RPT)])


def _k3(pack, table2):
    mesh = plsc.VectorSubcoreMesh(core_axis_name="c", subcore_axis_name="s")
    f = pl.kernel(
        _k3_body,
        out_type=jax.ShapeDtypeStruct((HEADS, NU, H), jnp.float32),
        mesh=mesh,
        scratch_types=[
            pltpu.VMEM((4 * CHUNK,), jnp.int32),
            pltpu.VMEM((4 * CHUNK,), jnp.int32),
            pltpu.VMEM((CHUNK,), jnp.int32),
            pltpu.VMEM((CHUNK,), jnp.int32),
            pltpu.VMEM((CHUNK, H), jnp.float32),
            pltpu.VMEM((CHUNK, H), jnp.float32),
            pltpu.SemaphoreType.DMA,
            pltpu.SemaphoreType.DMA,
            pltpu.SemaphoreType.DMA,
            pltpu.SemaphoreType.DMA,
            pltpu.SemaphoreType.DMA,
            pltpu.SemaphoreType.DMA,
            pltpu.VMEM_SHARED((NU, H), jnp.float32),
        ],
        compiler_params=pltpu.CompilerParams(needs_layout_passes=False),
    )
    return f(pack, table2)


def _edge_phase_jax(xf, src, dst, table, asv, adv):
    a_s = asv[:, 0, :][:, xf]
    a_d = adv[:, 0, :][:, xf]
    alpha = a_s[:, src] + a_d[:, dst]
    alpha = jnp.maximum(alpha, 0.2 * alpha)
    ex = jnp.exp(alpha)
    dn = jax.ops.segment_sum(ex.T, dst, num_segments=N)
    xs = xf[src]
    u0 = jax.ops.segment_sum(table[0][xs] * ex[0][:, None], dst, num_segments=N)
    u1 = jax.ops.segment_sum(table[1][xs] * ex[1][:, None], dst, num_segments=N)
    U = jnp.stack([u0, u1], axis=0)
    dnp = jnp.zeros((2, HEADS, N // NBLK, NBLK), jnp.float32)
    dnp = dnp.at[0].set(dn.T.reshape(HEADS, N // NBLK, NBLK))
    dnp = dnp.reshape(2 * HEADS, N // NBLK, NBLK).transpose(1, 0, 2)
    return U, dnp


def kernel(x, edge_index, batch_idx, emb, W, att_src, att_dst, bias_gat,
           ln_g, ln_b, lin_W, lin_b):
    xf = x.reshape(N).astype(jnp.int32)
    loop = jnp.arange(N, dtype=jnp.int32)
    pad = jnp.zeros(PE - NE, dtype=jnp.int32)
    src_p = jnp.concatenate([edge_index[0].astype(jnp.int32), loop, pad])
    dst_p = jnp.concatenate([edge_index[1].astype(jnp.int32), loop, pad])

    emb_p = jnp.pad(emb, ((0, VP - VOCAB), (0, 0)))
    att_src3 = att_src.reshape(HEADS, 1, H)
    att_dst3 = att_dst.reshape(HEADS, 1, H)
    table, asv, adv = _k1(emb_p, W, att_src3, att_dst3)

    asv2 = asv.reshape(HEADS * VP)
    adv2 = adv.reshape(HEADS * VP)
    pack, dnp_flat = _k2(xf, src_p, dst_p, asv2, adv2)
    table2 = table.reshape(HEADS * VP, H)
    U = _k3(pack, table2)

    dnp = dnp_flat.reshape(2, 2, NP_DN)[:, :, :N]
    dnp = dnp.reshape(2 * HEADS, N // NBLK, NBLK).transpose(1, 0, 2)

    bidx3 = batch_idx.astype(jnp.int32).reshape(N // NBLK, 1, NBLK)
    h, z = _k4(U, dnp, bidx3, bias_gat, ln_g, ln_b, lin_W, lin_b)
    return h, z
